# Initial kernel scaffold; baseline (speedup 1.0000x reference)
import jax
import jax.numpy as jnp
from jax.experimental import pallas as pl


def kernel(school_idx, grade_idx, goal_idx, subject_idx, method_idx, school_table, grade_table, goal_table, subject_table, method_table, W1, b1, W2, b2, W3, b3):
    raise NotImplementedError("write your pallas kernel here")


# speedup vs baseline: 1.7837x; 1.7837x over previous
import functools

import jax
import jax.numpy as jnp
from jax.experimental import pallas as pl
from jax.experimental.pallas import tpu as pltpu

B = 16384
EMB = 32
VSIZES = (52, 14, 12, 14, 8)
OFFS = (0, 52, 66, 78, 92)
VPAD = 128
BLK = 2048


def _mlp_body(idx_ref, t_ref, w1_ref, b1_ref, w2_ref, b2_ref, w3_ref, b3_ref,
              out_ref, m_ref):
    @pl.when(pl.program_id(0) == 0)
    def _fold():
        m_ref[...] = jnp.dot(t_ref[...], w1_ref[...],
                             preferred_element_type=jnp.float32)

    iota = jax.lax.broadcasted_iota(jnp.int32, (BLK, VPAD), 1)
    acc = None
    for f in range(5):
        hot = (iota == idx_ref[:, f:f + 1] + OFFS[f])
        acc = hot if acc is None else jnp.logical_or(acc, hot)
    a = acc.astype(jnp.float32)

    h1 = jnp.maximum(
        jnp.dot(a, m_ref[...], preferred_element_type=jnp.float32)
        + b1_ref[...], 0.0)
    h2 = jnp.maximum(
        jnp.dot(h1, w2_ref[...], preferred_element_type=jnp.float32)
        + b2_ref[...], 0.0)
    out_ref[...] = (jnp.dot(h2, w3_ref[...], preferred_element_type=jnp.float32)
                    + b3_ref[...])


@functools.partial(jax.jit, static_argnames=())
def kernel(school_idx, grade_idx, goal_idx, subject_idx, method_idx,
           school_table, grade_table, goal_table, subject_table, method_table,
           W1, b1, W2, b2, W3, b3):
    idxs = [school_idx, grade_idx, goal_idx, subject_idx, method_idx]
    tables = [school_table, grade_table, goal_table, subject_table,
              method_table]

    idx_mat = jnp.zeros((B, 8), jnp.int32)
    for f, ix in enumerate(idxs):
        idx_mat = idx_mat.at[:, f].set(ix.astype(jnp.int32))

    t_stack = jnp.zeros((VPAD, 5 * EMB), jnp.float32)
    for f, t in enumerate(tables):
        t_stack = jax.lax.dynamic_update_slice(t_stack, t,
                                               (OFFS[f], f * EMB))

    grid = B // BLK
    out = pl.pallas_call(
        _mlp_body,
        grid=(grid,),
        in_specs=[
            pl.BlockSpec((BLK, 8), lambda i: (i, 0)),
            pl.BlockSpec((VPAD, 5 * EMB), lambda i: (0, 0)),
            pl.BlockSpec((5 * EMB, 256), lambda i: (0, 0)),
            pl.BlockSpec((1, 256), lambda i: (0, 0)),
            pl.BlockSpec((256, 128), lambda i: (0, 0)),
            pl.BlockSpec((1, 128), lambda i: (0, 0)),
            pl.BlockSpec((128, 32), lambda i: (0, 0)),
            pl.BlockSpec((1, 32), lambda i: (0, 0)),
        ],
        out_specs=pl.BlockSpec((BLK, 32), lambda i: (i, 0)),
        out_shape=jax.ShapeDtypeStruct((B, 32), jnp.float32),
        scratch_shapes=[pltpu.VMEM((VPAD, 256), jnp.float32)],
        compiler_params=pltpu.CompilerParams(
            dimension_semantics=("arbitrary",)),
    )(idx_mat, t_stack, W1, b1.reshape(1, 256), W2, b2.reshape(1, 128),
      W3, b3.reshape(1, 32))
    return out


# speedup vs baseline: 16.3072x; 9.1426x over previous
import functools

import jax
import jax.numpy as jnp
from jax.experimental import pallas as pl
from jax.experimental.pallas import tpu as pltpu

B = 16384
EMB = 32
VSIZES = (52, 14, 12, 14, 8)
PV = (56, 16, 16, 16, 8)
POFF = (0, 56, 72, 88, 104)
VPAD = 128
BLK = 2048


def _body(si_ref, gi_ref, oi_ref, ui_ref, mi_ref,
          st_ref, gt_ref, ot_ref, ut_ref, mt_ref,
          w1_ref, b1_ref, w2_ref, b2_ref, w3_ref, b3_ref,
          out_ref, m_ref):
    @pl.when(pl.program_id(0) == 0)
    def _fold():
        m_ref[...] = jnp.zeros((VPAD, 256), jnp.float32)
        for f, t_ref in enumerate((st_ref, gt_ref, ot_ref, ut_ref, mt_ref)):
            t = t_ref[...]
            if PV[f] > VSIZES[f]:
                t = jnp.concatenate(
                    [t, jnp.zeros((PV[f] - VSIZES[f], EMB), jnp.float32)], 0)
            w1f = w1_ref[f * EMB:(f + 1) * EMB, :]
            m_ref[POFF[f]:POFF[f] + PV[f], :] = jnp.dot(
                t, w1f, preferred_element_type=jnp.float32)

    iota = jax.lax.broadcasted_iota(jnp.int32, (VPAD, BLK), 0)
    acc = None
    for f, i_ref in enumerate((si_ref, gi_ref, oi_ref, ui_ref, mi_ref)):
        hot = (iota == i_ref[0] + POFF[f])
        acc = hot if acc is None else jnp.logical_or(acc, hot)
    a_t = acc.astype(jnp.float32)

    h1 = jnp.maximum(
        jax.lax.dot_general(a_t, m_ref[...], (((0,), (0,)), ((), ())),
                            preferred_element_type=jnp.float32)
        + b1_ref[...], 0.0)
    h2 = jnp.maximum(
        jnp.dot(h1, w2_ref[...], preferred_element_type=jnp.float32)
        + b2_ref[...], 0.0)
    out_ref[...] = (jnp.dot(h2, w3_ref[...], preferred_element_type=jnp.float32)
                    + b3_ref[...])


@jax.jit
def kernel(school_idx, grade_idx, goal_idx, subject_idx, method_idx,
           school_table, grade_table, goal_table, subject_table, method_table,
           W1, b1, W2, b2, W3, b3):
    grid = B // BLK
    idxs = [i.astype(jnp.int32).reshape(grid, 1, BLK)
            for i in (school_idx, grade_idx, goal_idx, subject_idx,
                      method_idx)]
    idx_spec = pl.BlockSpec((1, 1, BLK), lambda i: (i, 0, 0))
    full = lambda s: pl.BlockSpec(s, lambda i: tuple(0 for _ in s))
    out = pl.pallas_call(
        _body,
        grid=(grid,),
        in_specs=[idx_spec] * 5 + [
            full((VSIZES[0], EMB)), full((VSIZES[1], EMB)),
            full((VSIZES[2], EMB)), full((VSIZES[3], EMB)),
            full((VSIZES[4], EMB)),
            full((5 * EMB, 256)), full((1, 256)),
            full((256, 128)), full((1, 128)),
            full((128, 32)), full((1, 32)),
        ],
        out_specs=pl.BlockSpec((BLK, 32), lambda i: (i, 0)),
        out_shape=jax.ShapeDtypeStruct((B, 32), jnp.float32),
        scratch_shapes=[pltpu.VMEM((VPAD, 256), jnp.float32)],
        compiler_params=pltpu.CompilerParams(
            dimension_semantics=("arbitrary",)),
    )(*idxs, school_table, grade_table, goal_table, subject_table,
      method_table, W1, b1.reshape(1, 256), W2, b2.reshape(1, 128),
      W3, b3.reshape(1, 32))
    return out
